# baseline (speedup 1.0000x reference)
import jax
import jax.numpy as jnp
from jax import lax
from jax.experimental import pallas as pl
from jax.experimental.pallas import tpu as pltpu

V = 100000
D = 128
C = 20
N = 128
CH = 6400
NCH = (V + CH - 1) // CH
PADV = NCH * CH
ROWS = 8
RW = PADV // ROWS
K = 4
LAST = V - (NCH - 1) * CH


def _w2_copy(w2_hbm, bufs, sems, c):
    rows = LAST if c == NCH - 1 else CH
    q = c % K
    return pltpu.make_async_copy(
        w2_hbm.at[pl.ds(c * CH, rows), :],
        bufs.at[q, pl.ds(0, rows), :],
        sems.at[q],
    )


def _fused_kernel(idx_ref, emb_ref, w1_ref, b1_ref, w2_ref, b2_ref,
                  out_ref, g_ref, bufs_ref, gsem, sems):
    for p in range(C):
        pltpu.make_async_copy(
            emb_ref.at[pl.ds(idx_ref[p], 1), :],
            g_ref.at[pl.ds(p, 1), :],
            gsem,
        ).start()
    for c in range(K):
        _w2_copy(w2_ref, bufs_ref, sems, c).start()

    for p in range(C):
        pltpu.make_async_copy(
            emb_ref.at[pl.ds(idx_ref[p], 1), :],
            g_ref.at[pl.ds(p, 1), :],
            gsem,
        ).wait()
    acc = b1_ref[...].astype(jnp.float32)
    for p in range(C):
        acc = acc + lax.dot_general(
            g_ref[pl.ds(p, 1), :],
            w1_ref[:, pl.ds(p * D, D)],
            (((1,), (1,)), ((), ())),
            preferred_element_type=jnp.float32,
        )
    h = jnp.maximum(acc, 0.0)

    per_row = RW // CH
    for c in range(NCH):
        _w2_copy(w2_ref, bufs_ref, sems, c).wait()
        r, o = c // per_row, (c % per_row) * CH
        lb = lax.dot_general(
            h,
            bufs_ref[c % K],
            (((1,), (1,)), ((), ())),
            preferred_element_type=jnp.float32,
        ) + b2_ref[pl.ds(r, 1), pl.ds(o, CH)]
        if c == NCH - 1:
            cols = c * CH + lax.broadcasted_iota(jnp.int32, (1, CH), 1)
            lb = jnp.where(cols < V, lb, -1e30)
        out_ref[pl.ds(r, 1), pl.ds(o, CH)] = lb
        if c + K < NCH:
            _w2_copy(w2_ref, bufs_ref, sems, c + K).start()

    scr = out_ref[...]
    m = jnp.max(jnp.max(scr, axis=1, keepdims=True), axis=0, keepdims=True)
    s = jnp.sum(jnp.sum(jnp.exp(scr - m), axis=1, keepdims=True),
                axis=0, keepdims=True)
    out_ref[...] = scr - (m + jnp.log(s))


def kernel(inputs, emb, W1, b1, W2, b2):
    b1r = b1.reshape(1, N)
    b2r = jnp.pad(b2.reshape(1, V), ((0, 0), (0, PADV - V)),
                  constant_values=-1e30).reshape(ROWS, RW)
    out = pl.pallas_call(
        _fused_kernel,
        in_specs=[
            pl.BlockSpec(memory_space=pltpu.MemorySpace.SMEM),
            pl.BlockSpec(memory_space=pltpu.MemorySpace.HBM),
            pl.BlockSpec((N, C * D), lambda: (0, 0)),
            pl.BlockSpec((1, N), lambda: (0, 0)),
            pl.BlockSpec(memory_space=pltpu.MemorySpace.HBM),
            pl.BlockSpec((ROWS, RW), lambda: (0, 0)),
        ],
        out_specs=pl.BlockSpec((ROWS, RW), lambda: (0, 0)),
        out_shape=jax.ShapeDtypeStruct((ROWS, RW), jnp.float32),
        scratch_shapes=[
            pltpu.VMEM((C, D), jnp.float32),
            pltpu.VMEM((K, CH, D), jnp.float32),
            pltpu.SemaphoreType.DMA,
            pltpu.SemaphoreType.DMA((K,)),
        ],
        compiler_params=pltpu.CompilerParams(
            vmem_limit_bytes=100 * 1024 * 1024,
        ),
    )(inputs, emb, W1, b1r, W2, b2r)
    return out.reshape(1, PADV)[:, :V]


# speedup vs baseline: 1.0424x; 1.0424x over previous
import jax
import jax.numpy as jnp
from jax import lax
from jax.experimental import pallas as pl
from jax.experimental.pallas import tpu as pltpu

V = 100000
D = 128
C = 20
N = 128
CH = 6400
NCH = (V + CH - 1) // CH
PADV = NCH * CH
K = 4
LAST = V - (NCH - 1) * CH


def _w2_copy(w2_hbm, bufs, sems, c):
    rows = LAST if c == NCH - 1 else CH
    q = c % K
    return pltpu.make_async_copy(
        w2_hbm.at[pl.ds(c * CH, rows), :],
        bufs.at[q, pl.ds(0, rows), :],
        sems.at[q],
    )


def _fused_kernel(idx_ref, emb_ref, w1_ref, b1_ref, w2_ref, b2_ref,
                  out_ref, g_ref, bufs_ref, gsem, sems):
    for p in range(C):
        pltpu.make_async_copy(
            emb_ref.at[pl.ds(idx_ref[p], 1), :],
            g_ref.at[pl.ds(p, 1), :],
            gsem,
        ).start()
    for c in range(K):
        _w2_copy(w2_ref, bufs_ref, sems, c).start()

    for p in range(C):
        pltpu.make_async_copy(
            emb_ref.at[pl.ds(idx_ref[p], 1), :],
            g_ref.at[pl.ds(p, 1), :],
            gsem,
        ).wait()
    acc = b1_ref[...].astype(jnp.float32)
    for p in range(C):
        acc = acc + lax.dot_general(
            g_ref[pl.ds(p, 1), :],
            w1_ref[:, pl.ds(p * D, D)],
            (((1,), (1,)), ((), ())),
            preferred_element_type=jnp.float32,
        )
    h = jnp.maximum(acc, 0.0)

    m_run = jnp.full((1, 1), -1e30, jnp.float32)
    s_run = jnp.zeros((1, 1), jnp.float32)
    for c in range(NCH):
        _w2_copy(w2_ref, bufs_ref, sems, c).wait()
        lb = lax.dot_general(
            h,
            bufs_ref[c % K],
            (((1,), (1,)), ((), ())),
            preferred_element_type=jnp.float32,
        ) + b2_ref[0:1, pl.ds(c * CH, CH)]
        if c == NCH - 1:
            cols = c * CH + lax.broadcasted_iota(jnp.int32, (1, CH), 1)
            lb = jnp.where(cols < V, lb, -1e30)
        out_ref[0:1, pl.ds(c * CH, CH)] = lb
        if c + K < NCH:
            _w2_copy(w2_ref, bufs_ref, sems, c + K).start()
        bm = jnp.max(lb, axis=1, keepdims=True)
        bs = jnp.sum(jnp.exp(lb - bm), axis=1, keepdims=True)
        m_new = jnp.maximum(m_run, bm)
        s_run = s_run * jnp.exp(m_run - m_new) + bs * jnp.exp(bm - m_new)
        m_run = m_new

    logz = m_run + jnp.log(s_run)
    out_ref[...] = out_ref[...] - logz


def kernel(inputs, emb, W1, b1, W2, b2):
    b1r = b1.reshape(1, N)
    b2r = jnp.pad(b2.reshape(1, V), ((0, 0), (0, PADV - V)),
                  constant_values=-1e30)
    out = pl.pallas_call(
        _fused_kernel,
        in_specs=[
            pl.BlockSpec(memory_space=pltpu.MemorySpace.SMEM),
            pl.BlockSpec(memory_space=pltpu.MemorySpace.HBM),
            pl.BlockSpec((N, C * D), lambda: (0, 0)),
            pl.BlockSpec((1, N), lambda: (0, 0)),
            pl.BlockSpec(memory_space=pltpu.MemorySpace.HBM),
            pl.BlockSpec((1, PADV), lambda: (0, 0)),
        ],
        out_specs=pl.BlockSpec((1, PADV), lambda: (0, 0)),
        out_shape=jax.ShapeDtypeStruct((1, PADV), jnp.float32),
        scratch_shapes=[
            pltpu.VMEM((C, D), jnp.float32),
            pltpu.VMEM((K, CH, D), jnp.float32),
            pltpu.SemaphoreType.DMA,
            pltpu.SemaphoreType.DMA((K,)),
        ],
        compiler_params=pltpu.CompilerParams(
            vmem_limit_bytes=100 * 1024 * 1024,
        ),
    )(inputs, emb, W1, b1r, W2, b2r)
    return out[:, :V]
